# Initial kernel scaffold; baseline (speedup 1.0000x reference)
import jax
import jax.numpy as jnp
from jax.experimental import pallas as pl


def kernel(gt_depth, pred_depth, fx, fy):
    raise NotImplementedError("write your pallas kernel here")


# speedup vs baseline: 12.2089x; 12.2089x over previous
import functools

import numpy as np
import jax
import jax.numpy as jnp
from jax import lax
from jax.experimental import pallas as pl
from jax.experimental.pallas import tpu as pltpu, tpu_sc as plsc

H = 384
W = 384
B = 4
HW = H * W
G = int(HW * 0.15)
LANES = 128
GP = 24576
N = B * GP
NROW = N // LANES
NW = 32
CHUNK = N // NW
ROWS = CHUNK // LANES

_DELTA_Z = 1e-4
_DELTA_COS = 0.867
_DXYZ = 0.005


def _static_consts():
    rng = np.random.RandomState(0)
    ps = []
    for _ in range(3):
        p = rng.choice(HW, G, replace=True)
        rng.shuffle(p)
        ps.append(p.astype(np.int64))
    idx = np.zeros((3, B, GP), np.int32)
    cu = np.zeros((3, GP), np.float32)
    cv = np.zeros((3, GP), np.float32)
    pm = np.zeros((GP,), np.float32)
    pm[:G] = 1.0
    for k in range(3):
        p = ps[k]
        cu[k, :G] = (p % W).astype(np.float32) - float(W // 2)
        cv[k, :G] = (p // W).astype(np.float32) - float(H // 2)
        for b in range(B):
            idx[k, b, :G] = (b * HW + p).astype(np.int32)
    cu_f = np.broadcast_to(cu[:, None, :], (3, B, GP)).reshape(3, NROW, LANES)
    cv_f = np.broadcast_to(cv[:, None, :], (3, B, GP)).reshape(3, NROW, LANES)
    pm_f = np.broadcast_to(pm[None, :], (B, GP)).reshape(NROW, LANES)
    return (idx.reshape(3, NROW, LANES), np.ascontiguousarray(cu_f),
            np.ascontiguousarray(cv_f), np.ascontiguousarray(pm_f))


_IDX, _CU, _CV, _PM = _static_consts()

@functools.lru_cache(maxsize=1)
def _get_sc_gather():
    mesh = plsc.VectorSubcoreMesh(core_axis_name="c", subcore_axis_name="s",
                                  num_cores=2, num_subcores=16)

    @functools.partial(
        pl.kernel,
        out_type=(jax.ShapeDtypeStruct((3, NROW, LANES), jnp.float32),
                  jax.ShapeDtypeStruct((3, NROW, LANES), jnp.float32)),
        mesh=mesh,
        scratch_types=[
            pltpu.VMEM((ROWS, LANES), jnp.int32),
            pltpu.VMEM((ROWS, LANES), jnp.float32),
            pltpu.VMEM((ROWS, LANES), jnp.float32),
            pltpu.SemaphoreType.DMA,
        ],
    )
    def _sc_gather(gt_hbm, pr_hbm, idx_hbm, outg, outp, idx_v, gbuf, pbuf, sem):
        wid = lax.axis_index("s") * 2 + lax.axis_index("c")
        rowbase = wid * ROWS
        for k in range(3):
            pltpu.sync_copy(idx_hbm.at[k, pl.ds(rowbase, ROWS)], idx_v)
            handles = []
            for r in range(ROWS):
                handles.append(
                    pltpu.async_copy(gt_hbm.at[idx_v.at[r]], gbuf.at[r], sem))
                handles.append(
                    pltpu.async_copy(pr_hbm.at[idx_v.at[r]], pbuf.at[r], sem))
            for h in handles:
                h.wait()
            pltpu.sync_copy(gbuf, outg.at[k, pl.ds(rowbase, ROWS)])
            pltpu.sync_copy(pbuf, outp.at[k, pl.ds(rowbase, ROWS)])

    return _sc_gather


def _loss_body(ggt, gpr, cx, cy, pm, out):
    dg = [ggt[k] for k in range(3)]
    dp = [gpr[k] for k in range(3)]
    cxs = [cx[k] for k in range(3)]
    cys = [cy[k] for k in range(3)]

    gx = [cxs[k] * jnp.abs(dg[k]) for k in range(3)]
    gy = [cys[k] * jnp.abs(dg[k]) for k in range(3)]
    gz = dg

    pX = [cxs[k] * jnp.abs(dp[k]) for k in range(3)]
    pY = [cys[k] * jnp.abs(dp[k]) for k in range(3)]
    pZ = list(dp)
    cz = [dp[j] == 0.0 for j in range(3)]
    pX = [jnp.where(cz[0], jnp.float32(1e-4), pX[k]) for k in range(3)]
    pY = [jnp.where(cz[1], jnp.float32(1e-4), pY[k]) for k in range(3)]
    pZ = [jnp.where(cz[2], jnp.float32(1e-4), pZ[k]) for k in range(3)]

    pairs = ((0, 1), (0, 2), (1, 2))
    e = [(gx[j] - gx[i], gy[j] - gy[i], gz[j] - gz[i]) for (i, j) in pairs]

    def dot(a, b):
        return a[0] * b[0] + a[1] * b[1] + a[2] * b[2]

    d00 = dot(e[0], e[0])
    d11 = dot(e[1], e[1])
    d22 = dot(e[2], e[2])
    d01 = dot(e[0], e[1])
    d02 = dot(e[0], e[2])
    d12 = dot(e[1], e[2])
    n0 = jnp.sqrt(d00)
    n1 = jnp.sqrt(d11)
    n2 = jnp.sqrt(d22)
    eps = jnp.float32(1e-8)

    def hit(num, na, nb):
        return (jnp.abs(num / (na * nb + eps)) > _DELTA_COS).astype(jnp.int32)

    cnt = (hit(d00, n0, n0) + hit(d11, n1, n1) + hit(d22, n2, n2)
           + 2 * hit(d01, n0, n1) + 2 * hit(d02, n0, n2) + 2 * hit(d12, n1, n2))
    mask_cos = cnt > 3
    mask_pad = (gz[0] > _DELTA_Z) & (gz[1] > _DELTA_Z) & (gz[2] > _DELTA_Z)
    mx = ((jnp.abs(e[0][0]) < _DXYZ) | (jnp.abs(e[1][0]) < _DXYZ)
          | (jnp.abs(e[2][0]) < _DXYZ))
    my = ((jnp.abs(e[0][1]) < _DXYZ) | (jnp.abs(e[1][1]) < _DXYZ)
          | (jnp.abs(e[2][1]) < _DXYZ))
    mz = ((jnp.abs(e[0][2]) < _DXYZ) | (jnp.abs(e[1][2]) < _DXYZ)
          | (jnp.abs(e[2][2]) < _DXYZ))
    mask = mask_pad & jnp.logical_not((mx & my & mz) | mask_cos) & (pm[...] > 0)

    def cross(a, b):
        return (a[1] * b[2] - a[2] * b[1],
                a[2] * b[0] - a[0] * b[2],
                a[0] * b[1] - a[1] * b[0])

    ng = cross(e[0], e[1])
    f0 = (pX[1] - pX[0], pY[1] - pY[0], pZ[1] - pZ[0])
    f1 = (pX[2] - pX[0], pY[2] - pY[0], pZ[2] - pZ[0])
    nd = cross(f0, f1)
    gn = jnp.sqrt(dot(ng, ng))
    dn = jnp.sqrt(dot(nd, nd))
    gn = gn + (gn == 0.0).astype(jnp.float32) * 0.01
    dn = dn + (dn == 0.0).astype(jnp.float32) * 0.01
    lo = (jnp.abs(ng[0] / gn - nd[0] / dn)
          + jnp.abs(ng[1] / gn - nd[1] / dn)
          + jnp.abs(ng[2] / gn - nd[2] / dn))

    vbits = jnp.where(mask, lax.bitcast_convert_type(lo, jnp.int32),
                      jnp.int32(-1))
    n_valid = jnp.sum(mask.astype(jnp.int32))
    n_drop = n_valid // 4
    count = n_valid - n_drop

    def step(t, ans):
        cand = ans | lax.shift_left(jnp.int32(1), jnp.int32(30) - t)
        c2 = jnp.sum((vbits >= cand).astype(jnp.int32))
        return jnp.where(c2 >= count, cand, ans)

    ans = lax.fori_loop(0, 31, step, jnp.int32(0))
    gtm = vbits > ans
    cnt_gt = jnp.sum(gtm.astype(jnp.int32))
    sum_gt = jnp.sum(jnp.where(gtm, lo, jnp.float32(0.0)))
    tstar = lax.bitcast_convert_type(ans, jnp.float32)
    kept = sum_gt + (count - cnt_gt).astype(jnp.float32) * tstar
    res = kept / count.astype(jnp.float32)
    out[...] = jnp.broadcast_to(res, (1, 1))


def kernel(gt_depth, pred_depth, fx, fy):
    gflat = gt_depth.reshape(-1)
    pflat = pred_depth.reshape(-1)
    gg, gp = _get_sc_gather()(gflat, pflat, jnp.asarray(_IDX))
    cx = jnp.asarray(_CU) / fx
    cy = jnp.asarray(_CV) / fy
    out = pl.pallas_call(
        _loss_body,
        out_shape=jax.ShapeDtypeStruct((1, 1), jnp.float32),
    )(gg, gp, cx, cy, jnp.asarray(_PM))
    return out[0, 0]
